# Initial kernel scaffold; baseline (speedup 1.0000x reference)
import jax
import jax.numpy as jnp
from jax.experimental import pallas as pl


def kernel(x, gate_w, w1, w2):
    raise NotImplementedError("write your pallas kernel here")


# speedup vs baseline: 5.9649x; 5.9649x over previous
import functools

import jax
import jax.numpy as jnp
from jax.experimental import pallas as pl
from jax.experimental.pallas import tpu as pltpu
from jax.experimental.pallas import tpu_sc as plsc

E = 64
D = 768
FF = 1024
N = 2048
T = 128
TM = N // T
STEPS = TM + E - 1
EPAD = 128
_BIG = jnp.int32(1 << 30)


def _gating_body(x_ref, gwt_ref, eid_ref):
    xb = x_ref[...]
    s = jnp.dot(xb, gwt_ref[...], preferred_element_type=jnp.float32)
    col = jax.lax.broadcasted_iota(jnp.int32, s.shape, 1)
    s = jnp.where(col < E, s, -jnp.inf)
    m = jnp.max(s, axis=1, keepdims=True)
    idx = jnp.min(jnp.where(s == m, col, E), axis=1)
    eid_ref[...] = idx.reshape(1, 1, T)


def _ffn_body(st_ref, se_ref, sv_ref, off_ref,
              x_ref, w1_ref, w2_ref, gwt_ref, out_ref):
    s = pl.program_id(0)
    t = st_ref[s]
    e = se_ref[s]
    valid = sv_ref[s]
    prev_t = st_ref[jnp.maximum(s - 1, 0)]
    first = jnp.logical_or(s == 0, prev_t != t)

    @pl.when(first)
    def _():
        out_ref[...] = jnp.zeros_like(out_ref)

    xb = x_ref[...]
    sc = jnp.dot(xb, gwt_ref[...], preferred_element_type=jnp.float32)
    col = jax.lax.broadcasted_iota(jnp.int32, sc.shape, 1)
    sc = jnp.where(col < E, sc, -jnp.inf)
    m = jnp.max(sc, axis=1, keepdims=True)
    w = 1.0 / jnp.sum(jnp.exp(sc - m), axis=1, keepdims=True)

    h = jnp.dot(xb, w1_ref[0], preferred_element_type=jnp.float32)
    h = h * jax.nn.sigmoid(h)
    y = jnp.dot(h, w2_ref[0], preferred_element_type=jnp.float32)

    rows = t * T + jax.lax.broadcasted_iota(jnp.int32, (T, 1), 0)
    mask = (rows >= off_ref[e]) & (rows < off_ref[e + 1]) & (valid > 0)
    out_ref[...] += jnp.where(mask, y * w, 0.0)


def _route_metadata(eid):
    eid = eid.astype(jnp.int32)
    sort_idx = jnp.argsort(eid).astype(jnp.int32)
    sorted_eid = jnp.sort(eid)
    inv_idx = jnp.argsort(sort_idx).astype(jnp.int32)
    offsets = jnp.searchsorted(
        sorted_eid, jnp.arange(E + 1, dtype=jnp.int32), side="left"
    ).astype(jnp.int32)

    t = jnp.arange(TM, dtype=jnp.int32)[:, None]
    e = jnp.arange(E, dtype=jnp.int32)[None, :]
    lo = offsets[:-1][None, :]
    hi = offsets[1:][None, :]
    valid = (lo < (t + 1) * T) & (hi > t * T) & (hi > lo)

    keys = jnp.where(valid, t * E + e, _BIG).reshape(-1)
    keys = jnp.sort(keys)[:STEPS]
    is_valid = keys < _BIG
    nv = jnp.sum(is_valid.astype(jnp.int32))
    raw_t = jnp.where(is_valid, keys // E, 0).astype(jnp.int32)
    raw_e = jnp.where(is_valid, keys % E, 0).astype(jnp.int32)
    last_t = jnp.take(raw_t, nv - 1)
    last_e = jnp.take(raw_e, nv - 1)
    sidx = jnp.arange(STEPS, dtype=jnp.int32)
    step_t = jnp.where(sidx < nv, raw_t, last_t)
    step_e = jnp.where(sidx < nv, raw_e, last_e)
    step_v = (sidx < nv).astype(jnp.int32)
    return sort_idx, inv_idx, offsets, step_t, step_e, step_v


def _sc_row_gather(table, idx):
    info = plsc.get_sparse_core_info()
    nw = info.num_cores * info.num_subcores
    bpw = N // nw
    mesh = plsc.VectorSubcoreMesh(core_axis_name="c", subcore_axis_name="s")

    @functools.partial(
        pl.kernel,
        mesh=mesh,
        out_type=jax.ShapeDtypeStruct((N, D), jnp.float32),
        scratch_types=[
            pltpu.VMEM((bpw,), jnp.int32),
            pltpu.VMEM((bpw, D), jnp.float32),
            pltpu.SemaphoreType.DMA,
        ],
    )
    def gather_k(table_hbm, idx_hbm, out_hbm, idx_v, rows_v, sem):
        wid = jax.lax.axis_index("s") * info.num_cores + jax.lax.axis_index("c")
        base = wid * bpw
        pltpu.sync_copy(idx_hbm.at[pl.ds(base, bpw)], idx_v)
        pltpu.async_copy(table_hbm.at[idx_v], rows_v, sem).wait()
        pltpu.sync_copy(rows_v, out_hbm.at[pl.ds(base, bpw)])

    return gather_k(table, idx)


def kernel(x, gate_w, w1, w2):
    orig_shape = x.shape
    xf = x.reshape(-1, x.shape[-1]).astype(jnp.float32)
    gwt = jnp.zeros((D, EPAD), jnp.float32).at[:, :E].set(gate_w.T)

    eid3 = pl.pallas_call(
        _gating_body,
        grid=(TM,),
        in_specs=[
            pl.BlockSpec((T, D), lambda t: (t, 0)),
            pl.BlockSpec((D, EPAD), lambda t: (0, 0)),
        ],
        out_specs=pl.BlockSpec((1, 1, T), lambda t: (t, 0, 0)),
        out_shape=jax.ShapeDtypeStruct((TM, 1, T), jnp.int32),
    )(xf, gwt)
    eid = eid3.reshape(N)

    sort_idx, inv_idx, offsets, step_t, step_e, step_v = _route_metadata(eid)

    x_sorted = _sc_row_gather(xf, sort_idx)

    grid_spec = pltpu.PrefetchScalarGridSpec(
        num_scalar_prefetch=4,
        grid=(STEPS,),
        in_specs=[
            pl.BlockSpec((T, D), lambda s, st, se, sv, off: (st[s], 0)),
            pl.BlockSpec((1, D, FF), lambda s, st, se, sv, off: (se[s], 0, 0)),
            pl.BlockSpec((1, FF, D), lambda s, st, se, sv, off: (se[s], 0, 0)),
            pl.BlockSpec((D, EPAD), lambda s, st, se, sv, off: (0, 0)),
        ],
        out_specs=pl.BlockSpec((T, D), lambda s, st, se, sv, off: (st[s], 0)),
    )
    out_sorted = pl.pallas_call(
        _ffn_body,
        grid_spec=grid_spec,
        out_shape=jax.ShapeDtypeStruct((N, D), jnp.float32),
        compiler_params=pltpu.CompilerParams(
            dimension_semantics=("arbitrary",),
        ),
    )(step_t, step_e, step_v, offsets, x_sorted, w1, w2, gwt)

    y = _sc_row_gather(out_sorted, inv_idx)
    return y.reshape(orig_shape)
